# Initial kernel scaffold; baseline (speedup 1.0000x reference)
import jax
import jax.numpy as jnp
from jax.experimental import pallas as pl


def kernel(x, edge_index, W1, b1, W2, b2):
    raise NotImplementedError("write your pallas kernel here")


# speedup vs baseline: 13.0912x; 13.0912x over previous
import functools

import jax
import jax.numpy as jnp
from jax import lax
from jax.experimental import pallas as pl
from jax.experimental.pallas import tpu as pltpu
from jax.experimental.pallas import tpu_sc as plsc

NC = 2
NS = 16
NW = NC * NS

C = 128

_MESH = dict(core_axis_name="c", subcore_axis_name="s", num_cores=NC,
             num_subcores=NS)


def _sc_degree(npad, nch):
    stripe = npad // NS

    @functools.partial(
        pl.kernel,
        out_type=jax.ShapeDtypeStruct((NC, npad), jnp.float32),
        mesh=plsc.VectorSubcoreMesh(**_MESH),
        scratch_types=[
            pltpu.VMEM((nch, C), jnp.int32),
            pltpu.VMEM((C,), jnp.float32),
            pltpu.VMEM_SHARED((npad,), jnp.float32),
        ],
    )
    def k(dst_hbm, zeros_hbm, out_hbm, idx_d, ones_v, cnt_sh):
        cid = lax.axis_index("c")
        sid = lax.axis_index("s")
        for q in range(C // 16):
            ones_v[pl.ds(q * 16, 16)] = jnp.ones((16,), jnp.float32)
        pltpu.sync_copy(zeros_hbm.at[pl.ds(0, stripe)],
                        cnt_sh.at[pl.ds(sid * stripe, stripe)])
        plsc.subcore_barrier()
        pltpu.sync_copy(dst_hbm.at[cid, sid], idx_d)

        def body(j, _):
            pltpu.sync_copy(ones_v, cnt_sh.at[idx_d.at[j]], add=True)
            return 0

        lax.fori_loop(0, nch, body, 0)
        plsc.subcore_barrier()
        pltpu.sync_copy(cnt_sh.at[pl.ds(sid * stripe, stripe)],
                        out_hbm.at[cid, pl.ds(sid * stripe, stripe)])

    return k


def _sc_scatter(npad, d, nch):
    stripe = npad // NS
    zrows = 128

    @functools.partial(
        pl.kernel,
        out_type=jax.ShapeDtypeStruct((NC, npad, d), jnp.float32),
        mesh=plsc.VectorSubcoreMesh(**_MESH),
        scratch_types=[
            pltpu.VMEM((nch, C), jnp.int32),
            pltpu.VMEM((nch, C), jnp.int32),
            pltpu.VMEM((C, d), jnp.float32),
            pltpu.VMEM_SHARED((npad, d), jnp.float32),
            pltpu.SemaphoreType.DMA,
        ],
    )
    def k(table_hbm, src_hbm, dst_hbm, zeros_hbm, out_hbm,
          idx_s, idx_d, rows, acc_sh, sem):
        cid = lax.axis_index("c")
        sid = lax.axis_index("s")
        for q in range(stripe // zrows):
            pltpu.sync_copy(
                zeros_hbm,
                acc_sh.at[pl.ds(sid * stripe + q * zrows, zrows)])
        plsc.subcore_barrier()
        pltpu.sync_copy(src_hbm.at[cid, sid], idx_s)
        pltpu.sync_copy(dst_hbm.at[cid, sid], idx_d)

        def body(j, _):
            pltpu.async_copy(table_hbm.at[idx_s.at[j]], rows, sem).wait()
            pltpu.sync_copy(rows, acc_sh.at[idx_d.at[j]], add=True)
            return 0

        lax.fori_loop(0, nch, body, 0)
        plsc.subcore_barrier()
        pltpu.sync_copy(acc_sh.at[pl.ds(sid * stripe, stripe)],
                        out_hbm.at[cid, pl.ds(sid * stripe, stripe)])

    return k


def _tc_k1(npad, d, br):
    def body(x_ref, w_ref, p0_ref, p1_ref, hws_ref, dinv_ref):
        deg = p0_ref[...] + p1_ref[...] + 1.0
        dinv = lax.rsqrt(deg)
        hw = jnp.dot(x_ref[...], w_ref[...], preferred_element_type=jnp.float32)
        hws_ref[...] = hw * dinv
        dinv_ref[...] = dinv

    return pl.pallas_call(
        body,
        grid=(npad // br,),
        in_specs=[
            pl.BlockSpec((br, d), lambda i: (i, 0)),
            pl.BlockSpec((d, d), lambda i: (0, 0)),
            pl.BlockSpec((br, 1), lambda i: (i, 0)),
            pl.BlockSpec((br, 1), lambda i: (i, 0)),
        ],
        out_specs=[
            pl.BlockSpec((br, d), lambda i: (i, 0)),
            pl.BlockSpec((br, 1), lambda i: (i, 0)),
        ],
        out_shape=[
            jax.ShapeDtypeStruct((npad, d), jnp.float32),
            jax.ShapeDtypeStruct((npad, 1), jnp.float32),
        ],
    )


def _tc_k2(npad, d, br):
    def body(a0_ref, a1_ref, hws_ref, dinv_ref, b_ref, w_ref, out_ref):
        dinv = dinv_ref[...]
        s = a0_ref[...] + a1_ref[...] + hws_ref[...]
        h1 = jnp.maximum(dinv * s + b_ref[...], 0.0)
        out_ref[...] = jnp.dot(h1, w_ref[...],
                               preferred_element_type=jnp.float32) * dinv

    return pl.pallas_call(
        body,
        grid=(npad // br,),
        in_specs=[
            pl.BlockSpec((br, d), lambda i: (i, 0)),
            pl.BlockSpec((br, d), lambda i: (i, 0)),
            pl.BlockSpec((br, d), lambda i: (i, 0)),
            pl.BlockSpec((br, 1), lambda i: (i, 0)),
            pl.BlockSpec((1, d), lambda i: (0, 0)),
            pl.BlockSpec((d, d), lambda i: (0, 0)),
        ],
        out_specs=pl.BlockSpec((br, d), lambda i: (i, 0)),
        out_shape=jax.ShapeDtypeStruct((npad, d), jnp.float32),
    )


def _tc_k3(npad, d, br):
    def body(a0_ref, a1_ref, hws_ref, dinv_ref, b_ref, out_ref):
        s = a0_ref[...] + a1_ref[...] + hws_ref[...]
        out_ref[...] = dinv_ref[...] * s + b_ref[...]

    return pl.pallas_call(
        body,
        grid=(npad // br,),
        in_specs=[
            pl.BlockSpec((br, d), lambda i: (i, 0)),
            pl.BlockSpec((br, d), lambda i: (i, 0)),
            pl.BlockSpec((br, d), lambda i: (i, 0)),
            pl.BlockSpec((br, 1), lambda i: (i, 0)),
            pl.BlockSpec((1, d), lambda i: (0, 0)),
        ],
        out_specs=pl.BlockSpec((br, d), lambda i: (i, 0)),
        out_shape=jax.ShapeDtypeStruct((npad, d), jnp.float32),
    )


def kernel(x, edge_index, W1, b1, W2, b2):
    n, d = x.shape
    e = edge_index.shape[1]

    npad = ((n + 511) // 512) * 512
    per_w = -(-e // NW)
    nch = -(-per_w // C)
    epad = NW * nch * C

    trash = npad - 1
    src = jnp.concatenate(
        [edge_index[0], jnp.full((epad - e,), trash, jnp.int32)])
    dst = jnp.concatenate(
        [edge_index[1], jnp.full((epad - e,), trash, jnp.int32)])
    src_r = src.reshape(NC, NS, nch, C)
    dst_r = dst.reshape(NC, NS, nch, C)

    xp = jnp.concatenate([x, jnp.zeros((npad - n, d), x.dtype)])
    z1 = jnp.zeros((npad // NS,), jnp.float32)
    zrows = jnp.zeros((128, d), jnp.float32)
    b1r = b1.reshape(1, d)
    b2r = b2.reshape(1, d)

    br = 512

    cnt = _sc_degree(npad, nch)(dst_r, z1)
    hw1s, dinv = _tc_k1(npad, d, br)(
        xp, W1, cnt[0].reshape(npad, 1), cnt[1].reshape(npad, 1))
    acc1 = _sc_scatter(npad, d, nch)(hw1s, src_r, dst_r, zrows)
    hw2s = _tc_k2(npad, d, br)(acc1[0], acc1[1], hw1s, dinv, b1r, W2)
    acc2 = _sc_scatter(npad, d, nch)(hw2s, src_r, dst_r, zrows)
    out = _tc_k3(npad, d, br)(acc2[0], acc2[1], hw2s, dinv, b2r)
    return out[:n]
